# baseline (speedup 1.0000x reference)
import functools

import jax
import jax.numpy as jnp
from jax import lax
from jax.experimental import pallas as pl
from jax.experimental.pallas import tpu as pltpu
from jax.experimental.pallas import tpu_sc as plsc

B = 16384
D = 64
C = 20
N = 100
K = C + N
KP = 128
RBLK = 128
NVREG = D // 16
DEPTH = 4


@functools.lru_cache(maxsize=None)
def _build_sc_kernel():
  info = plsc.get_sparse_core_info()
  nc, ns = info.num_cores, info.num_subcores
  nw = nc * ns
  rpw = B // nw
  nblk = rpw // RBLK
  mesh = plsc.VectorSubcoreMesh(core_axis_name="c", subcore_axis_name="s")

  scratch = (
      pltpu.VMEM((RBLK,), jnp.int32),
      pltpu.VMEM((RBLK, KP), jnp.int32),
      pltpu.VMEM((RBLK, D), jnp.float32),
      [pltpu.VMEM((K, D), jnp.float32) for _ in range(DEPTH)],
      pltpu.VMEM((RBLK, D), jnp.float32),
      pltpu.VMEM((RBLK, D), jnp.float32),
      [pltpu.SemaphoreType.DMA for _ in range(DEPTH)],
      pltpu.SemaphoreType.DMA,
  )

  @functools.partial(
      pl.kernel,
      out_type=(
          jax.ShapeDtypeStruct((B, D), jnp.float32),
          jax.ShapeDtypeStruct((B, D), jnp.float32),
          jax.ShapeDtypeStruct((B, D), jnp.float32),
      ),
      mesh=mesh,
      compiler_params=pltpu.CompilerParams(use_tc_tiling_on_sc=False),
      scratch_types=scratch,
  )
  def sc_kernel(center_hbm, idx_hbm, in_emb_hbm, out_emb_hbm,
                crows_o, ctxsum_o, negsum_o,
                cidx_v, idx_v, crows_v, bufs, ctxsum_v, negsum_v,
                sems, sem_c):
    wid = lax.axis_index("s") * nc + lax.axis_index("c")

    def issue(b, slot):
      pltpu.make_async_copy(
          out_emb_hbm.at[idx_v.at[b, pl.ds(0, K)]], bufs[slot],
          sems[slot]).start()

    def drain(slot):
      pltpu.make_async_copy(
          out_emb_hbm.at[idx_v.at[0, pl.ds(0, K)]], bufs[slot],
          sems[slot]).wait()

    def reduce_store(slot, b):
      buf = bufs[slot]
      for k in range(NVREG):
        sl = pl.ds(16 * k, 16)
        acc_c = buf[0, sl]
        for j in range(1, C):
          acc_c = acc_c + buf[j, sl]
        acc_n = buf[C, sl]
        for j in range(C + 1, K):
          acc_n = acc_n + buf[j, sl]
        ctxsum_v[b, sl] = acc_c
        negsum_v[b, sl] = acc_n

    def block(blk, carry):
      base = pl.multiple_of(wid * rpw + blk * RBLK, RBLK)
      pltpu.sync_copy(center_hbm.at[pl.ds(base, RBLK)], cidx_v)
      pltpu.sync_copy(idx_hbm.at[pl.ds(base, RBLK), :], idx_v)
      crows_cp = pltpu.make_async_copy(in_emb_hbm.at[cidx_v], crows_v, sem_c)
      crows_cp.start()

      for s in range(DEPTH - 1):
        issue(s, s)

      def group(g, c2):
        for s in range(DEPTH):
          b = g * DEPTH + s

          @pl.when(b + DEPTH - 1 < RBLK)
          def _():
            issue(b + DEPTH - 1, (s + DEPTH - 1) % DEPTH)

          drain(s)
          reduce_store(s, b)
        return c2

      lax.fori_loop(0, RBLK // DEPTH, group, 0)

      crows_cp.wait()
      pltpu.sync_copy(crows_v, crows_o.at[pl.ds(base, RBLK), :])
      pltpu.sync_copy(ctxsum_v, ctxsum_o.at[pl.ds(base, RBLK), :])
      pltpu.sync_copy(negsum_v, negsum_o.at[pl.ds(base, RBLK), :])
      return carry

    lax.fori_loop(0, nblk, block, 0)

  return sc_kernel


def _tc_format_idx(center, ctx, neg):
  bt = 2048

  def body(ctr_ref, c_ref, n_ref, o_ref, ctr_o_ref):
    pad = jnp.zeros((bt, KP - K), jnp.int32)
    o_ref[...] = jax.lax.concatenate([c_ref[...], n_ref[...], pad], 1)
    ctr_o_ref[...] = ctr_ref[...]

  return pl.pallas_call(
      body,
      grid=(B // bt,),
      in_specs=[
          pl.BlockSpec((bt,), lambda i: (i,)),
          pl.BlockSpec((bt, C), lambda i: (i, 0)),
          pl.BlockSpec((bt, N), lambda i: (i, 0)),
      ],
      out_specs=[
          pl.BlockSpec((bt, KP), lambda i: (i, 0)),
          pl.BlockSpec((bt,), lambda i: (i,)),
      ],
      out_shape=[
          jax.ShapeDtypeStruct((B, KP), jnp.int32),
          jax.ShapeDtypeStruct((B,), jnp.int32),
      ],
  )(center, ctx, neg)


def _tc_score(crows, ctxsum, negsum):
  bt = 2048

  def body(c_ref, cs_ref, ns_ref, o_ref):
    c = c_ref[...]
    s_ctx = jnp.sum(cs_ref[...] * c, axis=1)
    s_neg = jnp.sum(ns_ref[...] * c, axis=1)
    o_ref[...] = -(jax.nn.log_sigmoid(s_ctx) + jax.nn.log_sigmoid(-s_neg))

  return pl.pallas_call(
      body,
      grid=(B // bt,),
      in_specs=[pl.BlockSpec((bt, D), lambda i: (i, 0))] * 3,
      out_specs=pl.BlockSpec((bt,), lambda i: (i,)),
      out_shape=jax.ShapeDtypeStruct((B,), jnp.float32),
  )(crows, ctxsum, negsum)


def kernel(center_word_label, context_words_labels, neg_words_labels,
           input_embedding, output_embedding):
  idx_all, center = _tc_format_idx(
      center_word_label.astype(jnp.int32),
      context_words_labels.astype(jnp.int32),
      neg_words_labels.astype(jnp.int32))
  crows, ctxsum, negsum = _build_sc_kernel()(
      center, idx_all, input_embedding, output_embedding)
  return _tc_score(crows, ctxsum, negsum)


# speedup vs baseline: 1.6370x; 1.6370x over previous
import functools

import jax
import jax.numpy as jnp
from jax import lax
from jax.experimental import pallas as pl
from jax.experimental.pallas import tpu as pltpu
from jax.experimental.pallas import tpu_sc as plsc

B = 16384
D = 64
C = 20
N = 100
K = C + N
VOCAB = 1000000
RBLK = 128
NVREG = D // 16
DEPTH = 4
TRBW = 2048


def _tc_linearize(table):
  t_t = table.T

  def body(x_ref, o_ref):
    y = x_ref[...].T
    o_ref[...] = jax.lax.concatenate(
        [y, jnp.zeros((TRBW, 2 * D - D), jnp.float32)], 1)

  grid = (VOCAB + TRBW - 1) // TRBW
  return pl.pallas_call(
      body,
      grid=(grid,),
      in_specs=[pl.BlockSpec((D, TRBW), lambda j: (0, j))],
      out_specs=pl.BlockSpec((TRBW, 2 * D), lambda j: (j, 0)),
      out_shape=jax.ShapeDtypeStruct((VOCAB, 2 * D), jnp.float32),
  )(t_t)


@functools.lru_cache(maxsize=None)
def _build_sc_main():
  info = plsc.get_sparse_core_info()
  nc, ns = info.num_cores, info.num_subcores
  nw = nc * ns
  rpw = B // nw
  nblk = rpw // RBLK
  mesh = plsc.VectorSubcoreMesh(core_axis_name="c", subcore_axis_name="s")

  scratch = (
      pltpu.VMEM((RBLK, K), jnp.int32),
      [pltpu.VMEM((K, 2 * D), jnp.float32) for _ in range(DEPTH)],
      pltpu.VMEM((RBLK, D), jnp.float32),
      pltpu.VMEM((RBLK, D), jnp.float32),
      [pltpu.SemaphoreType.DMA for _ in range(DEPTH)],
  )

  @functools.partial(
      pl.kernel,
      out_type=(
          jax.ShapeDtypeStruct((B, D), jnp.float32),
          jax.ShapeDtypeStruct((B, D), jnp.float32),
      ),
      mesh=mesh,
      compiler_params=pltpu.CompilerParams(use_tc_tiling_on_sc=False),
      scratch_types=scratch,
  )
  def sc_main(idx_hbm, emb_hbm, ctxsum_o, negsum_o,
              idx_v, bufs, ctxsum_v, negsum_v, sems):
    wid = lax.axis_index("s") * nc + lax.axis_index("c")

    def issue(b, slot):
      pltpu.make_async_copy(
          emb_hbm.at[idx_v.at[b]], bufs[slot], sems[slot]).start()

    def drain(slot):
      pltpu.make_async_copy(
          emb_hbm.at[idx_v.at[0]], bufs[slot], sems[slot]).wait()

    def seg_sum(buf, lo, hi):
      sls = [pl.ds(16 * k, 16) for k in range(NVREG)]
      acc = [buf[lo, sls[k]] for k in range(NVREG)]
      unroll = 8
      n = hi - lo - 1
      rem = n % unroll

      def step(i, a):
        j0 = lo + 1 + i * unroll
        for u in range(unroll):
          a = [a[k] + buf[j0 + u, sls[k]] for k in range(NVREG)]
        return a

      acc = lax.fori_loop(0, n // unroll, step, acc)
      for j in range(hi - rem, hi):
        acc = [acc[k] + buf[j, sls[k]] for k in range(NVREG)]
      return acc

    def reduce_store(slot, b):
      buf = bufs[slot]
      acc_c = seg_sum(buf, 0, C)
      acc_n = seg_sum(buf, C, K)
      for k in range(NVREG):
        sl = pl.ds(16 * k, 16)
        ctxsum_v[b, sl] = acc_c[k]
        negsum_v[b, sl] = acc_n[k]

    def block(blk, carry):
      base = pl.multiple_of(wid * rpw + blk * RBLK, RBLK)
      pltpu.sync_copy(idx_hbm.at[pl.ds(base, RBLK), :], idx_v)

      for s in range(DEPTH - 1):
        issue(s, s)

      def group(g, c2):
        for s in range(DEPTH):
          b = g * DEPTH + s

          @pl.when(b + DEPTH - 1 < RBLK)
          def _():
            issue(b + DEPTH - 1, (s + DEPTH - 1) % DEPTH)

          drain(s)
          reduce_store(s, b)
        return c2

      lax.fori_loop(0, RBLK // DEPTH, group, 0)

      pltpu.sync_copy(ctxsum_v, ctxsum_o.at[pl.ds(base, RBLK), :])
      pltpu.sync_copy(negsum_v, negsum_o.at[pl.ds(base, RBLK), :])
      return carry

    lax.fori_loop(0, nblk, block, 0)

  return sc_main


@functools.lru_cache(maxsize=None)
def _build_sc_center():
  info = plsc.get_sparse_core_info()
  nc, ns = info.num_cores, info.num_subcores
  nw = nc * ns
  rpw = B // nw
  ngat = rpw // 128
  mesh = plsc.VectorSubcoreMesh(core_axis_name="c", subcore_axis_name="s")

  @functools.partial(
      pl.kernel,
      out_type=jax.ShapeDtypeStruct((B, 2 * D), jnp.float32),
      mesh=mesh,
      compiler_params=pltpu.CompilerParams(use_tc_tiling_on_sc=False),
      scratch_types=(
          pltpu.VMEM((B // 32,), jnp.int32),
          pltpu.VMEM((B // 32, 2 * D), jnp.float32),
          pltpu.SemaphoreType.DMA,
      ),
  )
  def sc_center(center_hbm, emb_hbm, crows_o, cidx_v, crows_v, sem):
    wid = lax.axis_index("s") * nc + lax.axis_index("c")
    base = pl.multiple_of(wid * rpw, rpw)
    pltpu.sync_copy(center_hbm.at[pl.ds(base, rpw)], cidx_v)
    for i in range(ngat):
      pltpu.make_async_copy(
          emb_hbm.at[cidx_v.at[pl.ds(i * 128, 128)]],
          crows_v.at[pl.ds(i * 128, 128), :], sem).start()
    for i in range(ngat):
      pltpu.make_async_copy(
          emb_hbm.at[cidx_v.at[pl.ds(0, 128)]],
          crows_v.at[pl.ds(0, 128), :], sem).wait()
    pltpu.sync_copy(crows_v, crows_o.at[pl.ds(base, rpw), :])

  return sc_center


def _tc_score(crows, ctxsum, negsum):
  bt = 2048

  def body(c_ref, cs_ref, ns_ref, o_ref):
    c = c_ref[:, :D]
    s_ctx = jnp.sum(cs_ref[...] * c, axis=1)
    s_neg = jnp.sum(ns_ref[...] * c, axis=1)
    o_ref[...] = -(jax.nn.log_sigmoid(s_ctx) + jax.nn.log_sigmoid(-s_neg))

  return pl.pallas_call(
      body,
      grid=(B // bt,),
      in_specs=[pl.BlockSpec((bt, 2 * D), lambda i: (i, 0))] +
               [pl.BlockSpec((bt, D), lambda i: (i, 0))] * 2,
      out_specs=pl.BlockSpec((bt,), lambda i: (i,)),
      out_shape=jax.ShapeDtypeStruct((B,), jnp.float32),
  )(crows, ctxsum, negsum)


def kernel(center_word_label, context_words_labels, neg_words_labels,
           input_embedding, output_embedding):
  idx_all = jnp.concatenate(
      [context_words_labels.astype(jnp.int32),
       neg_words_labels.astype(jnp.int32)], axis=1)
  out_lin = _tc_linearize(output_embedding)
  in_lin = _tc_linearize(input_embedding)
  ctxsum, negsum = _build_sc_main()(idx_all, out_lin)
  crows = _build_sc_center()(center_word_label.astype(jnp.int32), in_lin)
  return _tc_score(crows, ctxsum, negsum)
